# baseline (speedup 1.0000x reference)
import functools

import jax
import jax.numpy as jnp
from jax import lax
from jax.experimental import pallas as pl
from jax.experimental.pallas import tpu as pltpu
from jax.experimental.pallas import tpu_sc as plsc

N_NODES = 10000
DIM = 128
NUM_EDGES = 320000

_NC = 2
_NS = 16
_NW = _NC * _NS
_LANES = 16

_EPW = NUM_EDGES // _NW
_CHUNK = 112
_NCHUNKS = 91
_EPW_PAD = _CHUNK * _NCHUNKS
_GROUPS = _CHUNK // _LANES
_DPAIRS = DIM // 2
_UNROLL = 16


def _h_body(emb_ref, w_ref, b_ref, out_ref):
    acts = lax.dot_general(
        emb_ref[...], w_ref[...],
        dimension_numbers=(((1,), (1,)), ((), ())),
        preferred_element_type=jnp.float32,
    )
    out_ref[...] = jnp.maximum(acts + b_ref[...], 0.0).astype(jnp.bfloat16)


def _compute_h(emb, w0, b0):
    return pl.pallas_call(
        _h_body,
        out_shape=jax.ShapeDtypeStruct((N_NODES, DIM), jnp.bfloat16),
    )(emb, w0, b0)


_mesh = plsc.VectorSubcoreMesh(core_axis_name="c", subcore_axis_name="s")


@functools.partial(
    pl.kernel,
    mesh=_mesh,
    compiler_params=pltpu.CompilerParams(needs_layout_passes=False,
                                         use_tc_tiling_on_sc=False),
    out_type=jax.ShapeDtypeStruct((_NW, _LANES), jnp.float32),
    scratch_types=[
        pltpu.VMEM((_NCHUNKS, _CHUNK), jnp.int32),
        pltpu.VMEM((_NCHUNKS, _CHUNK), jnp.int32),
        pltpu.VMEM((_EPW_PAD,), jnp.float32),
        pltpu.VMEM((_CHUNK, _DPAIRS), jnp.int32),
        pltpu.VMEM((_CHUNK, _DPAIRS), jnp.int32),
        pltpu.VMEM((_CHUNK, _DPAIRS), jnp.int32),
        pltpu.VMEM((_CHUNK, _DPAIRS), jnp.int32),
        pltpu.VMEM((_LANES,), jnp.float32),
        pltpu.VMEM_SHARED((N_NODES, _DPAIRS), jnp.int32),
        pltpu.SemaphoreType.DMA,
        pltpu.SemaphoreType.DMA,
        pltpu.SemaphoreType.DMA,
        pltpu.SemaphoreType.DMA,
    ],
)
def _sc_loss(h_hbm, src_hbm, dst_hbm, lab_hbm, out_hbm,
             sidx_v, didx_v, lab_v, rows_sa, rows_da, rows_sb, rows_db,
             out_v, h_spm, sem_sa, sem_da, sem_sb, sem_db):
    wid = lax.axis_index("s") * _NC + lax.axis_index("c")
    lane_iota = lax.iota(jnp.int32, _LANES)

    @pl.when(lax.axis_index("s") == 0)
    def _():
        pltpu.sync_copy(h_hbm, h_spm)

    pltpu.sync_copy(src_hbm.at[wid], sidx_v)
    pltpu.sync_copy(dst_hbm.at[wid], didx_v)
    pltpu.sync_copy(lab_hbm.at[wid], lab_v)
    plsc.subcore_barrier()

    def issue(ci, rows_s, rows_d, sem_s, sem_d):
        pltpu.async_copy(h_spm.at[sidx_v.at[ci]], rows_s, sem_s)
        pltpu.async_copy(h_spm.at[didx_v.at[ci]], rows_d, sem_d)

    def wait(ci, rows_s, rows_d, sem_s, sem_d):
        pltpu.make_async_copy(h_spm.at[sidx_v.at[ci]], rows_s, sem_s).wait()
        pltpu.make_async_copy(h_spm.at[didx_v.at[ci]], rows_d, sem_d).wait()

    def compute_chunk(ci, rows_s, rows_d, loss_acc):
        def group_body(g, acc_in):
            row_ids = g * _LANES + lane_iota

            def dim_body(jo, carry):
                sq0, sq1, col = carry
                for _ in range(_UNROLL):
                    s2 = plsc.load_gather(rows_s, [row_ids, col])
                    d2 = plsc.load_gather(rows_d, [row_ids, col])
                    dd = (plsc.bitcast(s2, jnp.bfloat16)
                          - plsc.bitcast(d2, jnp.bfloat16))
                    dd2 = dd * dd
                    e0, e1 = plsc.unpack(dd2, format=plsc.PackFormat.INTERLEAVED)
                    sq0 = sq0 + e0
                    sq1 = sq1 + e1
                    col = (col + 1) & (_DPAIRS - 1)
                return sq0, sq1, col

            sq0, sq1, _ = lax.fori_loop(
                0, _DPAIRS // _UNROLL, dim_body,
                (jnp.zeros((_LANES,), jnp.float32),
                 jnp.zeros((_LANES,), jnp.float32),
                 lane_iota))
            predicts = jnp.exp((sq0 + sq1) * (-1.0 / DIM))
            lbl = lab_v[pl.ds(ci * _CHUNK + g * _LANES, _LANES)]
            err = lbl - predicts
            return acc_in + 0.5 * err * err

        return lax.fori_loop(0, _GROUPS, group_body, loss_acc)

    issue(0, rows_sa, rows_da, sem_sa, sem_da)

    def pair_body(p, loss_acc):
        ci_a = 2 * p
        issue(ci_a + 1, rows_sb, rows_db, sem_sb, sem_db)
        wait(ci_a, rows_sa, rows_da, sem_sa, sem_da)
        loss_acc = compute_chunk(ci_a, rows_sa, rows_da, loss_acc)
        issue(ci_a + 2, rows_sa, rows_da, sem_sa, sem_da)
        wait(ci_a + 1, rows_sb, rows_db, sem_sb, sem_db)
        return compute_chunk(ci_a + 1, rows_sb, rows_db, loss_acc)

    loss = lax.fori_loop(0, _NCHUNKS // 2, pair_body,
                         jnp.zeros((_LANES,), jnp.float32))
    wait(_NCHUNKS - 1, rows_sa, rows_da, sem_sa, sem_da)
    loss = compute_chunk(_NCHUNKS - 1, rows_sa, rows_da, loss)
    out_v[...] = loss
    pltpu.sync_copy(out_v, out_hbm.at[wid])


def kernel(edges, labels, embedding_state, W, b):
    pad = _EPW_PAD - _EPW
    src = jnp.pad(edges[:, 0].reshape(_NW, _EPW), ((0, 0), (0, pad)))
    dst = jnp.pad(edges[:, 1].reshape(_NW, _EPW), ((0, 0), (0, pad)))
    lab = jnp.pad(labels.reshape(_NW, _EPW), ((0, 0), (0, pad)),
                  constant_values=1.0)
    src = src.reshape(_NW, _NCHUNKS, _CHUNK)
    dst = dst.reshape(_NW, _NCHUNKS, _CHUNK)
    h16 = _compute_h(embedding_state, W[0], b[0].reshape(1, DIM))
    h_i32 = lax.bitcast_convert_type(
        h16.reshape(N_NODES, _DPAIRS, 2), jnp.int32)
    partials = _sc_loss(h_i32, src, dst, lab)
    return jnp.sum(partials)


# speedup vs baseline: 1.0198x; 1.0198x over previous
import functools

import jax
import jax.numpy as jnp
from jax import lax
from jax.experimental import pallas as pl
from jax.experimental.pallas import tpu as pltpu
from jax.experimental.pallas import tpu_sc as plsc

N_NODES = 10000
DIM = 128
NUM_EDGES = 320000

_NC = 2
_NS = 16
_NW = _NC * _NS
_LANES = 16

_EPW = NUM_EDGES // _NW
_CHUNK = 64
_NCHUNKS = 157
_EPW_PAD = _CHUNK * _NCHUNKS
_GROUPS = _CHUNK // _LANES
_DPAIRS = DIM // 2
_UNROLL = 16


def _h_body(emb_ref, w_ref, b_ref, out_ref):
    acts = lax.dot_general(
        emb_ref[...], w_ref[...],
        dimension_numbers=(((1,), (1,)), ((), ())),
        preferred_element_type=jnp.float32,
    )
    out_ref[...] = jnp.maximum(acts + b_ref[...], 0.0).astype(jnp.bfloat16)


def _compute_h(emb, w0, b0):
    return pl.pallas_call(
        _h_body,
        out_shape=jax.ShapeDtypeStruct((N_NODES, DIM), jnp.bfloat16),
    )(emb, w0, b0)


_mesh = plsc.VectorSubcoreMesh(core_axis_name="c", subcore_axis_name="s")


@functools.partial(
    pl.kernel,
    mesh=_mesh,
    compiler_params=pltpu.CompilerParams(needs_layout_passes=False,
                                         use_tc_tiling_on_sc=False),
    out_type=jax.ShapeDtypeStruct((_NW, _LANES), jnp.float32),
    scratch_types=[
        pltpu.VMEM((_NCHUNKS, _CHUNK), jnp.int32),
        pltpu.VMEM((_NCHUNKS, _CHUNK), jnp.int32),
        pltpu.VMEM((_EPW_PAD,), jnp.float32),
        pltpu.VMEM((_CHUNK, _DPAIRS), jnp.int32),
        pltpu.VMEM((_CHUNK, _DPAIRS), jnp.int32),
        pltpu.VMEM((_CHUNK, _DPAIRS), jnp.int32),
        pltpu.VMEM((_CHUNK, _DPAIRS), jnp.int32),
        pltpu.VMEM((_LANES,), jnp.float32),
        pltpu.VMEM_SHARED((N_NODES, _DPAIRS), jnp.int32),
        pltpu.SemaphoreType.DMA,
        pltpu.SemaphoreType.DMA,
        pltpu.SemaphoreType.DMA,
        pltpu.SemaphoreType.DMA,
    ],
)
def _sc_loss(h_hbm, src_hbm, dst_hbm, lab_hbm, out_hbm,
             sidx_v, didx_v, lab_v, rows_sa, rows_da, rows_sb, rows_db,
             out_v, h_spm, sem_sa, sem_da, sem_sb, sem_db):
    wid = lax.axis_index("s") * _NC + lax.axis_index("c")
    lane_iota = lax.iota(jnp.int32, _LANES)

    @pl.when(lax.axis_index("s") == 0)
    def _():
        pltpu.sync_copy(h_hbm, h_spm)

    pltpu.sync_copy(src_hbm.at[wid], sidx_v)
    pltpu.sync_copy(dst_hbm.at[wid], didx_v)
    pltpu.sync_copy(lab_hbm.at[wid], lab_v)
    plsc.subcore_barrier()

    def issue(ci, rows_s, rows_d, sem_s, sem_d):
        pltpu.async_copy(h_spm.at[sidx_v.at[ci]], rows_s, sem_s)
        pltpu.async_copy(h_spm.at[didx_v.at[ci]], rows_d, sem_d)

    def wait(ci, rows_s, rows_d, sem_s, sem_d):
        pltpu.make_async_copy(h_spm.at[sidx_v.at[ci]], rows_s, sem_s).wait()
        pltpu.make_async_copy(h_spm.at[didx_v.at[ci]], rows_d, sem_d).wait()

    def compute_chunk(ci, rows_s, rows_d, loss_acc):
        def group_body(g, acc_in):
            row_ids = g * _LANES + lane_iota

            def dim_body(jo, carry):
                sq0, sq1, col = carry
                for _ in range(_UNROLL):
                    s2 = plsc.load_gather(rows_s, [row_ids, col])
                    d2 = plsc.load_gather(rows_d, [row_ids, col])
                    dd = (plsc.bitcast(s2, jnp.bfloat16)
                          - plsc.bitcast(d2, jnp.bfloat16))
                    dd2 = dd * dd
                    e0, e1 = plsc.unpack(dd2, format=plsc.PackFormat.INTERLEAVED)
                    sq0 = sq0 + e0
                    sq1 = sq1 + e1
                    col = (col + 1) & (_DPAIRS - 1)
                return sq0, sq1, col

            sq0, sq1, _ = lax.fori_loop(
                0, _DPAIRS // _UNROLL, dim_body,
                (jnp.zeros((_LANES,), jnp.float32),
                 jnp.zeros((_LANES,), jnp.float32),
                 lane_iota))
            predicts = jnp.exp((sq0 + sq1) * (-1.0 / DIM))
            lbl = lab_v[pl.ds(ci * _CHUNK + g * _LANES, _LANES)]
            err = lbl - predicts
            return acc_in + 0.5 * err * err

        return lax.fori_loop(0, _GROUPS, group_body, loss_acc)

    issue(0, rows_sa, rows_da, sem_sa, sem_da)

    def pair_body(p, loss_acc):
        ci_a = 2 * p
        issue(ci_a + 1, rows_sb, rows_db, sem_sb, sem_db)
        wait(ci_a, rows_sa, rows_da, sem_sa, sem_da)
        loss_acc = compute_chunk(ci_a, rows_sa, rows_da, loss_acc)
        issue(ci_a + 2, rows_sa, rows_da, sem_sa, sem_da)
        wait(ci_a + 1, rows_sb, rows_db, sem_sb, sem_db)
        return compute_chunk(ci_a + 1, rows_sb, rows_db, loss_acc)

    loss = lax.fori_loop(0, _NCHUNKS // 2, pair_body,
                         jnp.zeros((_LANES,), jnp.float32))
    wait(_NCHUNKS - 1, rows_sa, rows_da, sem_sa, sem_da)
    loss = compute_chunk(_NCHUNKS - 1, rows_sa, rows_da, loss)
    out_v[...] = loss
    pltpu.sync_copy(out_v, out_hbm.at[wid])


def kernel(edges, labels, embedding_state, W, b):
    pad = _EPW_PAD - _EPW
    src = jnp.pad(edges[:, 0].reshape(_NW, _EPW), ((0, 0), (0, pad)))
    dst = jnp.pad(edges[:, 1].reshape(_NW, _EPW), ((0, 0), (0, pad)))
    lab = jnp.pad(labels.reshape(_NW, _EPW), ((0, 0), (0, pad)),
                  constant_values=1.0)
    src = src.reshape(_NW, _NCHUNKS, _CHUNK)
    dst = dst.reshape(_NW, _NCHUNKS, _CHUNK)
    h16 = _compute_h(embedding_state, W[0], b[0].reshape(1, DIM))
    h_i32 = lax.bitcast_convert_type(
        h16.reshape(N_NODES, _DPAIRS, 2), jnp.int32)
    partials = _sc_loss(h_i32, src, dst, lab)
    return jnp.sum(partials)


# speedup vs baseline: 1.0649x; 1.0442x over previous
import functools

import jax
import jax.numpy as jnp
from jax import lax
from jax.experimental import pallas as pl
from jax.experimental.pallas import tpu as pltpu
from jax.experimental.pallas import tpu_sc as plsc

N_NODES = 10000
DIM = 128
NUM_EDGES = 320000

_NC = 2
_NS = 16
_NW = _NC * _NS
_LANES = 16

_EPW = NUM_EDGES // _NW
_CHUNK = 80
_NCHUNKS = 125
_EPW_PAD = _CHUNK * _NCHUNKS
_GROUPS = _CHUNK // _LANES
_DPAIRS = DIM // 2
_UNROLL = 16


def _h_body(emb_ref, w_ref, b_ref, out_ref):
    acts = lax.dot_general(
        emb_ref[...], w_ref[...],
        dimension_numbers=(((1,), (1,)), ((), ())),
        preferred_element_type=jnp.float32,
    )
    out_ref[...] = jnp.maximum(acts + b_ref[...], 0.0).astype(jnp.bfloat16)


def _compute_h(emb, w0, b0):
    return pl.pallas_call(
        _h_body,
        out_shape=jax.ShapeDtypeStruct((N_NODES, DIM), jnp.bfloat16),
    )(emb, w0, b0)


_mesh = plsc.VectorSubcoreMesh(core_axis_name="c", subcore_axis_name="s")


@functools.partial(
    pl.kernel,
    mesh=_mesh,
    compiler_params=pltpu.CompilerParams(needs_layout_passes=False,
                                         use_tc_tiling_on_sc=False),
    out_type=jax.ShapeDtypeStruct((_NW, _LANES), jnp.float32),
    scratch_types=[
        pltpu.VMEM((_NCHUNKS, _CHUNK), jnp.int32),
        pltpu.VMEM((_NCHUNKS, _CHUNK), jnp.int32),
        pltpu.VMEM((_EPW_PAD,), jnp.float32),
        pltpu.VMEM((_CHUNK, _DPAIRS), jnp.int32),
        pltpu.VMEM((_CHUNK, _DPAIRS), jnp.int32),
        pltpu.VMEM((_CHUNK, _DPAIRS), jnp.int32),
        pltpu.VMEM((_CHUNK, _DPAIRS), jnp.int32),
        pltpu.VMEM((_LANES,), jnp.float32),
        pltpu.VMEM_SHARED((N_NODES, _DPAIRS), jnp.int32),
        pltpu.SemaphoreType.DMA,
        pltpu.SemaphoreType.DMA,
        pltpu.SemaphoreType.DMA,
        pltpu.SemaphoreType.DMA,
    ],
)
def _sc_loss(h_hbm, src_hbm, dst_hbm, lab_hbm, out_hbm,
             sidx_v, didx_v, lab_v, rows_sa, rows_da, rows_sb, rows_db,
             out_v, h_spm, sem_sa, sem_da, sem_sb, sem_db):
    wid = lax.axis_index("s") * _NC + lax.axis_index("c")
    lane_iota = lax.iota(jnp.int32, _LANES)

    @pl.when(lax.axis_index("s") == 0)
    def _():
        pltpu.sync_copy(h_hbm, h_spm)

    pltpu.sync_copy(src_hbm.at[wid], sidx_v)
    pltpu.sync_copy(dst_hbm.at[wid], didx_v)
    pltpu.sync_copy(lab_hbm.at[wid], lab_v)
    plsc.subcore_barrier()

    def issue(ci, rows_s, rows_d, sem_s, sem_d):
        pltpu.async_copy(h_spm.at[sidx_v.at[ci]], rows_s, sem_s)
        pltpu.async_copy(h_spm.at[didx_v.at[ci]], rows_d, sem_d)

    def wait(ci, rows_s, rows_d, sem_s, sem_d):
        pltpu.make_async_copy(h_spm.at[sidx_v.at[ci]], rows_s, sem_s).wait()
        pltpu.make_async_copy(h_spm.at[didx_v.at[ci]], rows_d, sem_d).wait()

    def compute_chunk(ci, rows_s, rows_d, loss_acc):
        def group_body(g, acc_in):
            row_ids = g * _LANES + lane_iota

            def dim_body(jo, carry):
                sq0, sq1, col = carry
                for _ in range(_UNROLL):
                    s2 = plsc.load_gather(rows_s, [row_ids, col])
                    d2 = plsc.load_gather(rows_d, [row_ids, col])
                    dd = (plsc.bitcast(s2, jnp.bfloat16)
                          - plsc.bitcast(d2, jnp.bfloat16))
                    dd2 = dd * dd
                    e0, e1 = plsc.unpack(dd2, format=plsc.PackFormat.INTERLEAVED)
                    sq0 = sq0 + e0
                    sq1 = sq1 + e1
                    col = (col + 1) & (_DPAIRS - 1)
                return sq0, sq1, col

            sq0, sq1, _ = lax.fori_loop(
                0, _DPAIRS // _UNROLL, dim_body,
                (jnp.zeros((_LANES,), jnp.float32),
                 jnp.zeros((_LANES,), jnp.float32),
                 lane_iota))
            predicts = jnp.exp((sq0 + sq1) * (-1.0 / DIM))
            lbl = lab_v[pl.ds(ci * _CHUNK + g * _LANES, _LANES)]
            err = lbl - predicts
            return acc_in + 0.5 * err * err

        return lax.fori_loop(0, _GROUPS, group_body, loss_acc)

    issue(0, rows_sa, rows_da, sem_sa, sem_da)

    def pair_body(p, loss_acc):
        ci_a = 2 * p
        issue(ci_a + 1, rows_sb, rows_db, sem_sb, sem_db)
        wait(ci_a, rows_sa, rows_da, sem_sa, sem_da)
        loss_acc = compute_chunk(ci_a, rows_sa, rows_da, loss_acc)
        issue(ci_a + 2, rows_sa, rows_da, sem_sa, sem_da)
        wait(ci_a + 1, rows_sb, rows_db, sem_sb, sem_db)
        return compute_chunk(ci_a + 1, rows_sb, rows_db, loss_acc)

    loss = lax.fori_loop(0, _NCHUNKS // 2, pair_body,
                         jnp.zeros((_LANES,), jnp.float32))
    wait(_NCHUNKS - 1, rows_sa, rows_da, sem_sa, sem_da)
    loss = compute_chunk(_NCHUNKS - 1, rows_sa, rows_da, loss)
    out_v[...] = loss
    pltpu.sync_copy(out_v, out_hbm.at[wid])


def kernel(edges, labels, embedding_state, W, b):
    pad = _EPW_PAD - _EPW
    src = jnp.pad(edges[:, 0].reshape(_NW, _EPW), ((0, 0), (0, pad)))
    dst = jnp.pad(edges[:, 1].reshape(_NW, _EPW), ((0, 0), (0, pad)))
    lab = jnp.pad(labels.reshape(_NW, _EPW), ((0, 0), (0, pad)),
                  constant_values=1.0)
    src = src.reshape(_NW, _NCHUNKS, _CHUNK)
    dst = dst.reshape(_NW, _NCHUNKS, _CHUNK)
    h16 = _compute_h(embedding_state, W[0], b[0].reshape(1, DIM))
    h_i32 = lax.bitcast_convert_type(
        h16.reshape(N_NODES, _DPAIRS, 2), jnp.int32)
    partials = _sc_loss(h_i32, src, dst, lab)
    return jnp.sum(partials)
